# Initial kernel scaffold; baseline (speedup 1.0000x reference)
import jax
import jax.numpy as jnp
from jax.experimental import pallas as pl


def kernel(x, edge_index, W1_l, b1, W1_r, W2_l, b2, W2_r, W3_l, b3, W3_r):
    raise NotImplementedError("write your pallas kernel here")


# speedup vs baseline: 3.9770x; 3.9770x over previous
import functools

import jax
import jax.numpy as jnp
from jax import lax
from jax.experimental import pallas as pl
from jax.experimental.pallas import tpu as pltpu
from jax.experimental.pallas import tpu_sc as plsc

N_NODES = 10000
N_EDGES = 320000
D_IN = 128
D_HID = 128
N_CLASSES = 64

NCORES = 2
NTILES = 16
NW = NCORES * NTILES
NP = 10240
SLAB = NP // NTILES
CH = 128
NCH = 79
EPW = NCH * CH
E_PAD = EPW * NW
RB = 2048
DW = 128

_mesh = plsc.VectorSubcoreMesh(core_axis_name="c", subcore_axis_name="s")


def _make_sc_agg(d):
    def body(y_hbm, src_hbm, dst_hbm, zeros_hbm, out_hbm,
             acc, sidx, didx, rows, gsem, ssem):
        c = lax.axis_index("c")
        s = lax.axis_index("s")
        wid = s * NCORES + c
        pltpu.sync_copy(zeros_hbm, acc.at[pl.ds(s * SLAB, SLAB)])
        plsc.subcore_barrier()
        base = wid * EPW

        def step(j, carry):
            off = base + j * CH
            pltpu.sync_copy(src_hbm.at[pl.ds(off, CH)], sidx)
            pltpu.sync_copy(dst_hbm.at[pl.ds(off, CH)], didx)
            pltpu.async_copy(y_hbm.at[sidx], rows, gsem).wait()
            pltpu.async_copy(rows, acc.at[didx], ssem, add=True).wait()
            return carry

        lax.fori_loop(0, NCH, step, 0)
        plsc.subcore_barrier()
        pltpu.sync_copy(acc.at[pl.ds(s * SLAB, SLAB)],
                        out_hbm.at[c, pl.ds(s * SLAB, SLAB)])

    return pl.kernel(
        body,
        out_type=jax.ShapeDtypeStruct((NCORES, NP, d), jnp.float32),
        mesh=_mesh,
        scratch_types=[
            pltpu.VMEM_SHARED((NP, d), jnp.float32),
            pltpu.VMEM((CH,), jnp.int32),
            pltpu.VMEM((CH,), jnp.int32),
            pltpu.VMEM((CH, d), jnp.float32),
            pltpu.SemaphoreType.DMA,
            pltpu.SemaphoreType.DMA,
        ],
    )


def _make_sc_deg():
    def body(dst_hbm, ones_hbm, zeros_hbm, out_hbm, acc, didx, ones_v, ssem):
        c = lax.axis_index("c")
        s = lax.axis_index("s")
        wid = s * NCORES + c
        pltpu.sync_copy(zeros_hbm, acc.at[pl.ds(s * SLAB, SLAB)])
        pltpu.sync_copy(ones_hbm, ones_v)
        plsc.subcore_barrier()
        base = wid * EPW

        def step(j, carry):
            off = base + j * CH
            pltpu.sync_copy(dst_hbm.at[pl.ds(off, CH)], didx)
            pltpu.async_copy(ones_v, acc.at[didx], ssem, add=True).wait()
            return carry

        lax.fori_loop(0, NCH, step, 0)
        plsc.subcore_barrier()
        pltpu.sync_copy(acc.at[pl.ds(s * SLAB, SLAB)],
                        out_hbm.at[c, pl.ds(s * SLAB, SLAB)])

    return pl.kernel(
        body,
        out_type=jax.ShapeDtypeStruct((NCORES, NP, DW), jnp.float32),
        mesh=_mesh,
        scratch_types=[
            pltpu.VMEM_SHARED((NP, DW), jnp.float32),
            pltpu.VMEM((CH,), jnp.int32),
            pltpu.VMEM((CH, DW), jnp.float32),
            pltpu.SemaphoreType.DMA,
        ],
    )


def _tc_linear(x, wl_t, wr_t, b):
    n, din = x.shape
    dout = wl_t.shape[1]

    def body(x_ref, wl_ref, wr_ref, b_ref, y_ref, s_ref):
        xb = x_ref[...]
        y_ref[...] = jnp.dot(xb, wl_ref[...], preferred_element_type=jnp.float32)
        s_ref[...] = (jnp.dot(xb, wr_ref[...], preferred_element_type=jnp.float32)
                      + b_ref[...])

    return pl.pallas_call(
        body,
        grid=(n // RB,),
        in_specs=[
            pl.BlockSpec((RB, din), lambda i: (i, 0)),
            pl.BlockSpec((din, dout), lambda i: (0, 0)),
            pl.BlockSpec((din, dout), lambda i: (0, 0)),
            pl.BlockSpec((1, dout), lambda i: (0, 0)),
        ],
        out_specs=[
            pl.BlockSpec((RB, dout), lambda i: (i, 0)),
            pl.BlockSpec((RB, dout), lambda i: (i, 0)),
        ],
        out_shape=[
            jax.ShapeDtypeStruct((n, dout), jnp.float32),
            jax.ShapeDtypeStruct((n, dout), jnp.float32),
        ],
    )(x, wl_t, wr_t, b)


def _tc_combine_project(agg, deg, s_in, wl_t, wr_t, b):
    n, d = s_in.shape
    dout = wl_t.shape[1]

    def body(agg_ref, deg_ref, s_ref, wl_ref, wr_ref, b_ref, y_ref, s2_ref):
        a = agg_ref[0] + agg_ref[1]
        dg = deg_ref[0][:, 0:1] + deg_ref[1][:, 0:1]
        inv = 1.0 / jnp.maximum(dg, 1.0)
        h = jnp.maximum(a * inv + s_ref[...], 0.0)
        y_ref[...] = jnp.dot(h, wl_ref[...], preferred_element_type=jnp.float32)
        s2_ref[...] = (jnp.dot(h, wr_ref[...], preferred_element_type=jnp.float32)
                       + b_ref[...])

    return pl.pallas_call(
        body,
        grid=(n // RB,),
        in_specs=[
            pl.BlockSpec((NCORES, RB, d), lambda i: (0, i, 0)),
            pl.BlockSpec((NCORES, RB, DW), lambda i: (0, i, 0)),
            pl.BlockSpec((RB, d), lambda i: (i, 0)),
            pl.BlockSpec((d, dout), lambda i: (0, 0)),
            pl.BlockSpec((d, dout), lambda i: (0, 0)),
            pl.BlockSpec((1, dout), lambda i: (0, 0)),
        ],
        out_specs=[
            pl.BlockSpec((RB, dout), lambda i: (i, 0)),
            pl.BlockSpec((RB, dout), lambda i: (i, 0)),
        ],
        out_shape=[
            jax.ShapeDtypeStruct((n, dout), jnp.float32),
            jax.ShapeDtypeStruct((n, dout), jnp.float32),
        ],
    )(agg, deg, s_in, wl_t, wr_t, b)


def _tc_combine_logsoftmax(agg, deg, s_in):
    n, d = s_in.shape

    def body(agg_ref, deg_ref, s_ref, o_ref):
        a = agg_ref[0] + agg_ref[1]
        dg = deg_ref[0][:, 0:1] + deg_ref[1][:, 0:1]
        inv = 1.0 / jnp.maximum(dg, 1.0)
        z = (a * inv + s_ref[...])[:, :N_CLASSES]
        z = z - jnp.max(z, axis=1, keepdims=True)
        o_ref[...] = z - jnp.log(jnp.sum(jnp.exp(z), axis=1, keepdims=True))

    return pl.pallas_call(
        body,
        grid=(n // RB,),
        in_specs=[
            pl.BlockSpec((NCORES, RB, d), lambda i: (0, i, 0)),
            pl.BlockSpec((NCORES, RB, DW), lambda i: (0, i, 0)),
            pl.BlockSpec((RB, d), lambda i: (i, 0)),
        ],
        out_specs=pl.BlockSpec((RB, N_CLASSES), lambda i: (i, 0)),
        out_shape=jax.ShapeDtypeStruct((n, N_CLASSES), jnp.float32),
    )(agg, deg, s_in)


_sc_agg_hid = _make_sc_agg(D_HID)
_sc_deg = _make_sc_deg()


def kernel(x, edge_index, W1_l, b1, W1_r, W2_l, b2, W2_r, W3_l, b3, W3_r):
    x = x.astype(jnp.float32)
    src = edge_index[0].astype(jnp.int32)
    dst = edge_index[1].astype(jnp.int32)
    src_p = jnp.concatenate([src, jnp.zeros((E_PAD - N_EDGES,), jnp.int32)])
    dst_p = jnp.concatenate([dst, jnp.full((E_PAD - N_EDGES,), N_NODES, jnp.int32)])
    x_p = jnp.pad(x, ((0, NP - N_NODES), (0, 0)))

    zeros_hid = jnp.zeros((SLAB, D_HID), jnp.float32)
    zeros_deg = jnp.zeros((SLAB, DW), jnp.float32)
    ones_deg = jnp.ones((CH, DW), jnp.float32)

    W3_l_t = jnp.pad(W3_l.T, ((0, 0), (0, D_HID - N_CLASSES)))
    W3_r_t = jnp.pad(W3_r.T, ((0, 0), (0, D_HID - N_CLASSES)))
    b3_p = jnp.pad(b3[None, :], ((0, 0), (0, D_HID - N_CLASSES)))

    deg = _sc_deg(dst_p, ones_deg, zeros_deg)

    y1, s1 = _tc_linear(x_p, W1_l.T, W1_r.T, b1[None, :])
    agg1 = _sc_agg_hid(y1, src_p, dst_p, zeros_hid)
    y2, s2 = _tc_combine_project(agg1, deg, s1, W2_l.T, W2_r.T, b2[None, :])
    agg2 = _sc_agg_hid(y2, src_p, dst_p, zeros_hid)
    y3, s3 = _tc_combine_project(agg2, deg, s2, W3_l_t, W3_r_t, b3_p)
    agg3 = _sc_agg_hid(y3, src_p, dst_p, zeros_hid)
    out = _tc_combine_logsoftmax(agg3, deg, s3)
    return out[:N_NODES]
